# baseline (speedup 1.0000x reference)
import functools

import jax
import jax.numpy as jnp
from jax import lax
from jax.experimental import pallas as pl
from jax.experimental.pallas import tpu as pltpu
from jax.experimental.pallas import tpu_sc as plsc

H = 512
W = 512
C = 768
P = H * W
NW = 32
PIX_PER_W = P // NW
NCHUNK = PIX_PER_W // 16
NROUND = NCHUNK // 4


def _stage_indices(idx_ref, w_ref, flow_ref, first_pixel, slot):
  iota = lax.iota(jnp.int32, 16)
  p = first_pixel + iota
  col = lax.rem(p, W)
  row = lax.div(p, W)
  fx = plsc.load_gather(flow_ref, [iota * 2 + slot * 32])
  fy = plsc.load_gather(flow_ref, [iota * 2 + 1 + slot * 32])
  qx = col.astype(jnp.float32) + fx
  qy = row.astype(jnp.float32) + fy
  qxc = jnp.minimum(jnp.maximum(qx, 0.0), float(W - 2))
  qyc = jnp.minimum(jnp.maximum(qy, 0.0), float(H - 2))
  x0 = qxc.astype(jnp.int32)
  y0 = qyc.astype(jnp.int32)
  ax = jnp.minimum(jnp.maximum(qx - x0.astype(jnp.float32), 0.0), 1.0)
  ay = jnp.minimum(jnp.maximum(qy - y0.astype(jnp.float32), 0.0), 1.0)
  lin = y0 * W + x0
  idx_ref[pl.ds(0, 16)] = lin
  idx_ref[pl.ds(16, 16)] = lin + 1
  idx_ref[pl.ds(32, 16)] = lin + W
  idx_ref[pl.ds(48, 16)] = lin + (W + 1)
  w_ref[pl.ds(slot * 32, 16)] = ax
  w_ref[pl.ds(slot * 32 + 16, 16)] = ay


def _blend_chunk(corners_ref, w_ref, slot, out_ref):
  iota = lax.iota(jnp.int32, 16)
  ie = iota * 2
  io = ie + 1

  @plsc.parallel_loop(0, 16)
  def _pixel(pp):
    lane = lax.broadcast(pp, (16,))
    axs = plsc.load_gather(w_ref, [lane + slot * 32])
    ays = plsc.load_gather(w_ref, [lane + (slot * 32 + 16)])
    for k in range(C // 32):
      sl = pl.ds(k * 16, 16)
      tle, tlo = plsc.unpack(plsc.bitcast(corners_ref[pp, sl], jnp.bfloat16),
                             format=plsc.PackFormat.INTERLEAVED)
      tre, tro = plsc.unpack(plsc.bitcast(corners_ref[pp + 16, sl],
                                          jnp.bfloat16),
                             format=plsc.PackFormat.INTERLEAVED)
      ble, blo = plsc.unpack(plsc.bitcast(corners_ref[pp + 32, sl],
                                          jnp.bfloat16),
                             format=plsc.PackFormat.INTERLEAVED)
      bre, bro = plsc.unpack(plsc.bitcast(corners_ref[pp + 48, sl],
                                          jnp.bfloat16),
                             format=plsc.PackFormat.INTERLEAVED)
      tope = tle + axs * (tre - tle)
      bote = ble + axs * (bre - ble)
      topo = tlo + axs * (tro - tlo)
      boto = blo + axs * (bro - blo)
      plsc.store_scatter(out_ref, [lane, ie + k * 32],
                         tope + ays * (bote - tope))
      plsc.store_scatter(out_ref, [lane, io + k * 32],
                         topo + ays * (boto - topo))


def _warp_body(tab, flow, out, flow_v, w_v, idx0, idx1, idx2, idx3,
               c0, c1, c2, c3, o0, o1,
               gsem0, gsem1, gsem2, gsem3, osem0, osem1, fsem):
  wid = lax.axis_index("s") * 2 + lax.axis_index("c")
  base = wid * PIX_PER_W
  cbufs = (c0, c1, c2, c3)
  ibufs = (idx0, idx1, idx2, idx3)
  gsems = (gsem0, gsem1, gsem2, gsem3)
  obufs = (o0, o1)
  osems = (osem0, osem1)

  def flow_issue(q, slot):
    pltpu.async_copy(flow.at[pl.ds((base + q * 16) * 2, 32)],
                     flow_v.at[pl.ds(slot * 32, 32)], fsem)

  def flow_wait():
    pltpu.make_async_copy(flow.at[pl.ds(base * 2, 32)],
                          flow_v.at[pl.ds(0, 32)], fsem).wait()

  for s in range(4):
    flow_issue(s, s)
  for s in range(4):
    flow_wait()
    _stage_indices(ibufs[s], w_v, flow_v, base + s * 16, s)
    pltpu.async_copy(tab.at[ibufs[s]], cbufs[s], gsems[s])
    flow_issue(s + 4, s)

  def body(r, carry):
    for s in range(4):
      q = r * 4 + s
      pltpu.make_async_copy(tab.at[ibufs[s]], cbufs[s], gsems[s]).wait()

      @pl.when(q > 1)
      def _():
        pltpu.make_async_copy(obufs[s % 2], out.at[pl.ds(base, 16)],
                              osems[s % 2]).wait()

      _blend_chunk(cbufs[s], w_v, s, obufs[s % 2])
      pltpu.async_copy(obufs[s % 2], out.at[pl.ds(base + q * 16, 16)],
                       osems[s % 2])

      @pl.when(q + 4 < NCHUNK)
      def _():
        flow_wait()
        _stage_indices(ibufs[s], w_v, flow_v, base + (q + 4) * 16, s)
        pltpu.async_copy(tab.at[ibufs[s]], cbufs[s], gsems[s])

      @pl.when(q + 8 < NCHUNK)
      def _():
        flow_issue(q + 8, s)

    return carry

  lax.fori_loop(0, NROUND, body, 0)

  pltpu.make_async_copy(o0, out.at[pl.ds(base, 16)], osem0).wait()
  pltpu.make_async_copy(o1, out.at[pl.ds(base, 16)], osem1).wait()


def kernel(x, flow):
  B, h, w, c = x.shape
  assert (B, h, w, c) == (1, H, W, C) and flow.shape == (1, H, W, 2)
  xb = x.reshape(P, C // 2, 2).astype(jnp.bfloat16)
  tab = lax.bitcast_convert_type(xb, jnp.int32)
  flow_flat = flow.reshape(P * 2)

  warp = functools.partial(
      pl.kernel,
      out_type=jax.ShapeDtypeStruct((P, C), jnp.float32),
      mesh=plsc.VectorSubcoreMesh(core_axis_name="c", subcore_axis_name="s"),
      compiler_params=pltpu.CompilerParams(needs_layout_passes=False),
      scratch_types=[
          pltpu.VMEM((128,), jnp.float32),
          pltpu.VMEM((128,), jnp.float32),
          pltpu.VMEM((64,), jnp.int32),
          pltpu.VMEM((64,), jnp.int32),
          pltpu.VMEM((64,), jnp.int32),
          pltpu.VMEM((64,), jnp.int32),
          pltpu.VMEM((64, C // 2), jnp.int32),
          pltpu.VMEM((64, C // 2), jnp.int32),
          pltpu.VMEM((64, C // 2), jnp.int32),
          pltpu.VMEM((64, C // 2), jnp.int32),
          pltpu.VMEM((16, C), jnp.float32),
          pltpu.VMEM((16, C), jnp.float32),
          pltpu.SemaphoreType.DMA,
          pltpu.SemaphoreType.DMA,
          pltpu.SemaphoreType.DMA,
          pltpu.SemaphoreType.DMA,
          pltpu.SemaphoreType.DMA,
          pltpu.SemaphoreType.DMA,
          pltpu.SemaphoreType.DMA,
      ],
  )(_warp_body)

  out = warp(tab, flow_flat)
  return out.reshape(1, H, W, C)


# speedup vs baseline: 2.3099x; 2.3099x over previous
import functools

import jax
import jax.numpy as jnp
from jax import lax
from jax.experimental import pallas as pl
from jax.experimental.pallas import tpu as pltpu
from jax.experimental.pallas import tpu_sc as plsc

H = 512
W = 512
C = 768
P = H * W
NW = 32
PIX_PER_W = P // NW
NCHUNK = PIX_PER_W // 16
NROUND = NCHUNK // 4


def _stage_indices(idx_ref, w_ref, flow_ref, first_pixel, slot):
  iota = lax.iota(jnp.int32, 16)
  p = first_pixel + iota
  col = lax.rem(p, W)
  row = lax.div(p, W)
  fx = plsc.load_gather(flow_ref, [iota * 2 + slot * 32])
  fy = plsc.load_gather(flow_ref, [iota * 2 + 1 + slot * 32])
  qx = col.astype(jnp.float32) + fx
  qy = row.astype(jnp.float32) + fy
  qxc = jnp.minimum(jnp.maximum(qx, 0.0), float(W - 2))
  qyc = jnp.minimum(jnp.maximum(qy, 0.0), float(H - 2))
  x0 = qxc.astype(jnp.int32)
  y0 = qyc.astype(jnp.int32)
  ax = jnp.minimum(jnp.maximum(qx - x0.astype(jnp.float32), 0.0), 1.0)
  ay = jnp.minimum(jnp.maximum(qy - y0.astype(jnp.float32), 0.0), 1.0)
  lin = y0 * W + x0
  idx_ref[pl.ds(0, 16)] = lin
  idx_ref[pl.ds(16, 16)] = lin + 1
  idx_ref[pl.ds(32, 16)] = lin + W
  idx_ref[pl.ds(48, 16)] = lin + (W + 1)
  w_ref[pl.ds(slot * 32, 16)] = ax
  w_ref[pl.ds(slot * 32 + 16, 16)] = ay


def _blend_chunk(corners_ref, w_ref, slot, out_ref):
  @plsc.parallel_loop(0, 16)
  def _pixel(pp):
    lane = lax.broadcast(pp, (16,))
    axs = plsc.load_gather(w_ref, [lane + slot * 32])
    ays = plsc.load_gather(w_ref, [lane + (slot * 32 + 16)])
    for k in range(C // 32):
      sl = pl.ds(k * 16, 16)
      tle, tlo = plsc.unpack(plsc.bitcast(corners_ref[pp, sl], jnp.bfloat16),
                             format=plsc.PackFormat.INTERLEAVED)
      tre, tro = plsc.unpack(plsc.bitcast(corners_ref[pp + 16, sl],
                                          jnp.bfloat16),
                             format=plsc.PackFormat.INTERLEAVED)
      ble, blo = plsc.unpack(plsc.bitcast(corners_ref[pp + 32, sl],
                                          jnp.bfloat16),
                             format=plsc.PackFormat.INTERLEAVED)
      bre, bro = plsc.unpack(plsc.bitcast(corners_ref[pp + 48, sl],
                                          jnp.bfloat16),
                             format=plsc.PackFormat.INTERLEAVED)
      tope = tle + axs * (tre - tle)
      bote = ble + axs * (bre - ble)
      topo = tlo + axs * (tro - tlo)
      boto = blo + axs * (bro - blo)
      out_ref[pp, pl.ds(k * 16, 16)] = tope + ays * (bote - tope)
      out_ref[pp, pl.ds(C // 2 + k * 16, 16)] = topo + ays * (boto - topo)


def _warp_body(tab, flow, out, flow_v, w_v, idx0, idx1, idx2, idx3,
               c0, c1, c2, c3, o0, o1,
               gsem0, gsem1, gsem2, gsem3, osem0, osem1, fsem):
  wid = lax.axis_index("s") * 2 + lax.axis_index("c")
  base = wid * PIX_PER_W
  cbufs = (c0, c1, c2, c3)
  ibufs = (idx0, idx1, idx2, idx3)
  gsems = (gsem0, gsem1, gsem2, gsem3)
  obufs = (o0, o1)
  osems = (osem0, osem1)

  def flow_issue(q, slot):
    pltpu.async_copy(flow.at[pl.ds((base + q * 16) * 2, 32)],
                     flow_v.at[pl.ds(slot * 32, 32)], fsem)

  def flow_wait():
    pltpu.make_async_copy(flow.at[pl.ds(base * 2, 32)],
                          flow_v.at[pl.ds(0, 32)], fsem).wait()

  for s in range(4):
    flow_issue(s, s)
  for s in range(4):
    flow_wait()
    _stage_indices(ibufs[s], w_v, flow_v, base + s * 16, s)
    pltpu.async_copy(tab.at[ibufs[s]], cbufs[s], gsems[s])
    flow_issue(s + 4, s)

  def body(r, carry):
    for s in range(4):
      q = r * 4 + s
      pltpu.make_async_copy(tab.at[ibufs[s]], cbufs[s], gsems[s]).wait()

      @pl.when(q > 1)
      def _():
        pltpu.make_async_copy(obufs[s % 2], out.at[pl.ds(base, 16)],
                              osems[s % 2]).wait()

      _blend_chunk(cbufs[s], w_v, s, obufs[s % 2])
      pltpu.async_copy(obufs[s % 2], out.at[pl.ds(base + q * 16, 16)],
                       osems[s % 2])

      @pl.when(q + 4 < NCHUNK)
      def _():
        flow_wait()
        _stage_indices(ibufs[s], w_v, flow_v, base + (q + 4) * 16, s)
        pltpu.async_copy(tab.at[ibufs[s]], cbufs[s], gsems[s])

      @pl.when(q + 8 < NCHUNK)
      def _():
        flow_issue(q + 8, s)

    return carry

  lax.fori_loop(0, NROUND, body, 0)

  pltpu.make_async_copy(o0, out.at[pl.ds(base, 16)], osem0).wait()
  pltpu.make_async_copy(o1, out.at[pl.ds(base, 16)], osem1).wait()


def kernel(x, flow):
  B, h, w, c = x.shape
  assert (B, h, w, c) == (1, H, W, C) and flow.shape == (1, H, W, 2)
  u = lax.bitcast_convert_type(x.reshape(P, C), jnp.uint32)
  ua, ub = u[:, : C // 2], u[:, C // 2:]
  rne = lambda v: (v + 0x7FFF + ((v >> 16) & 1)) >> 16
  packed = rne(ua) | (rne(ub) << 16)
  tab = lax.bitcast_convert_type(packed, jnp.int32)
  flow_flat = flow.reshape(P * 2)

  warp = functools.partial(
      pl.kernel,
      out_type=jax.ShapeDtypeStruct((P, C), jnp.float32),
      mesh=plsc.VectorSubcoreMesh(core_axis_name="c", subcore_axis_name="s"),
      compiler_params=pltpu.CompilerParams(needs_layout_passes=False),
      scratch_types=[
          pltpu.VMEM((128,), jnp.float32),
          pltpu.VMEM((128,), jnp.float32),
          pltpu.VMEM((64,), jnp.int32),
          pltpu.VMEM((64,), jnp.int32),
          pltpu.VMEM((64,), jnp.int32),
          pltpu.VMEM((64,), jnp.int32),
          pltpu.VMEM((64, C // 2), jnp.int32),
          pltpu.VMEM((64, C // 2), jnp.int32),
          pltpu.VMEM((64, C // 2), jnp.int32),
          pltpu.VMEM((64, C // 2), jnp.int32),
          pltpu.VMEM((16, C), jnp.float32),
          pltpu.VMEM((16, C), jnp.float32),
          pltpu.SemaphoreType.DMA,
          pltpu.SemaphoreType.DMA,
          pltpu.SemaphoreType.DMA,
          pltpu.SemaphoreType.DMA,
          pltpu.SemaphoreType.DMA,
          pltpu.SemaphoreType.DMA,
          pltpu.SemaphoreType.DMA,
      ],
  )(_warp_body)

  out = warp(tab, flow_flat)
  return out.reshape(1, H, W, C)


# speedup vs baseline: 3.3796x; 1.4631x over previous
import functools

import jax
import jax.numpy as jnp
from jax import lax
from jax.experimental import pallas as pl
from jax.experimental.pallas import tpu as pltpu
from jax.experimental.pallas import tpu_sc as plsc

H = 512
W = 512
C = 768
P = H * W
NW = 32
PIX_PER_W = P // NW
NITER = PIX_PER_W // 32


def _stage_indices(idx_ref, w_ref, flow_ref, first_pixel, half):
  iota = lax.iota(jnp.int32, 16)
  p = first_pixel + iota
  col = lax.rem(p, W)
  row = lax.div(p, W)
  fx = plsc.load_gather(flow_ref, [iota * 2 + half * 32])
  fy = plsc.load_gather(flow_ref, [iota * 2 + 1 + half * 32])
  qx = col.astype(jnp.float32) + fx
  qy = row.astype(jnp.float32) + fy
  qxc = jnp.minimum(jnp.maximum(qx, 0.0), float(W - 2))
  qyc = jnp.minimum(jnp.maximum(qy, 0.0), float(H - 2))
  x0 = qxc.astype(jnp.int32)
  y0 = qyc.astype(jnp.int32)
  ax = jnp.minimum(jnp.maximum(qx - x0.astype(jnp.float32), 0.0), 1.0)
  ay = jnp.minimum(jnp.maximum(qy - y0.astype(jnp.float32), 0.0), 1.0)
  lin = y0 * W + x0
  idx_ref[pl.ds(0, 16)] = lin
  idx_ref[pl.ds(16, 16)] = lin + 1
  idx_ref[pl.ds(32, 16)] = lin + W
  idx_ref[pl.ds(48, 16)] = lin + (W + 1)
  w_ref[pl.ds(32 * half, 16)] = ax
  w_ref[pl.ds(32 * half + 16, 16)] = ay


def _blend_chunk(corners_ref, w_ref, half, out_ref):
  @plsc.parallel_loop(0, 16)
  def _pixel(pp):
    lane = lax.broadcast(pp, (16,))
    axs = plsc.load_gather(w_ref, [lane + 32 * half])
    ays = plsc.load_gather(w_ref, [lane + (32 * half + 16)])
    for k in range(C // 16):
      sl = pl.ds(k * 16, 16)
      tl = corners_ref[pp, sl]
      tr = corners_ref[pp + 16, sl]
      bl = corners_ref[pp + 32, sl]
      br = corners_ref[pp + 48, sl]
      top = tl + axs * (tr - tl)
      bot = bl + axs * (br - bl)
      out_ref[pp, sl] = top + ays * (bot - top)


def _warp_body(tab, flow, out, flow_v, w_v, idx0, idx1, c0, c1, o0, o1,
               gsem0, gsem1, osem0, osem1, fsem):
  wid = lax.axis_index("s") * 2 + lax.axis_index("c")
  base = wid * PIX_PER_W

  pltpu.sync_copy(flow.at[pl.ds(base * 2, 64)], flow_v)
  _stage_indices(idx0, w_v, flow_v, base, 0)
  pltpu.async_copy(tab.at[idx0], c0, gsem0)
  _stage_indices(idx1, w_v, flow_v, base + 16, 1)
  pltpu.async_copy(tab.at[idx1], c1, gsem1)

  def body(j, carry):
    nxt = j + 1
    not_last = nxt < NITER

    @pl.when(not_last)
    def _():
      pltpu.async_copy(flow.at[pl.ds((base + nxt * 32) * 2, 64)], flow_v,
                       fsem)

    pltpu.make_async_copy(tab.at[idx0], c0, gsem0).wait()

    @pl.when(j > 0)
    def _():
      pltpu.make_async_copy(o0, out.at[pl.ds(base, 16)], osem0).wait()

    row0 = base + j * 32
    _blend_chunk(c0, w_v, 0, o0)
    pltpu.async_copy(o0, out.at[pl.ds(row0, 16)], osem0)

    @pl.when(not_last)
    def _():
      pltpu.make_async_copy(flow.at[pl.ds(base * 2, 64)], flow_v, fsem).wait()
      _stage_indices(idx0, w_v, flow_v, base + nxt * 32, 0)
      pltpu.async_copy(tab.at[idx0], c0, gsem0)

    pltpu.make_async_copy(tab.at[idx1], c1, gsem1).wait()

    @pl.when(j > 0)
    def _():
      pltpu.make_async_copy(o1, out.at[pl.ds(base, 16)], osem1).wait()

    _blend_chunk(c1, w_v, 1, o1)
    pltpu.async_copy(o1, out.at[pl.ds(row0 + 16, 16)], osem1)

    @pl.when(not_last)
    def _():
      _stage_indices(idx1, w_v, flow_v, base + nxt * 32 + 16, 1)
      pltpu.async_copy(tab.at[idx1], c1, gsem1)

    return carry

  lax.fori_loop(0, NITER, body, 0)

  pltpu.make_async_copy(o0, out.at[pl.ds(base, 16)], osem0).wait()
  pltpu.make_async_copy(o1, out.at[pl.ds(base, 16)], osem1).wait()


def kernel(x, flow):
  B, h, w, c = x.shape
  assert (B, h, w, c) == (1, H, W, C) and flow.shape == (1, H, W, 2)
  tab = x.reshape(P, C)
  flow_flat = flow.reshape(P * 2)

  warp = functools.partial(
      pl.kernel,
      out_type=jax.ShapeDtypeStruct((P, C), jnp.float32),
      mesh=plsc.VectorSubcoreMesh(core_axis_name="c", subcore_axis_name="s"),
      compiler_params=pltpu.CompilerParams(needs_layout_passes=False),
      scratch_types=[
          pltpu.VMEM((64,), jnp.float32),
          pltpu.VMEM((64,), jnp.float32),
          pltpu.VMEM((64,), jnp.int32),
          pltpu.VMEM((64,), jnp.int32),
          pltpu.VMEM((64, C), jnp.float32),
          pltpu.VMEM((64, C), jnp.float32),
          pltpu.VMEM((16, C), jnp.float32),
          pltpu.VMEM((16, C), jnp.float32),
          pltpu.SemaphoreType.DMA,
          pltpu.SemaphoreType.DMA,
          pltpu.SemaphoreType.DMA,
          pltpu.SemaphoreType.DMA,
          pltpu.SemaphoreType.DMA,
      ],
  )(_warp_body)

  out = warp(tab, flow_flat)
  return out.reshape(1, H, W, C)
